# Initial kernel scaffold; baseline (speedup 1.0000x reference)
import jax
import jax.numpy as jnp
from jax.experimental import pallas as pl


def kernel(logits_ar, logits_parallel):
    raise NotImplementedError("write your pallas kernel here")


# speedup vs baseline: 70.6004x; 70.6004x over previous
import functools

import jax
import jax.numpy as jnp
from jax.experimental import pallas as pl
from jax.experimental.pallas import tpu as pltpu

_V = 100000
_C = 2048
_NC = (_V + _C - 1) // _C
_R = 64
_NEG = float("-inf")
_BIG = jnp.iinfo(jnp.int32).max


def _argmax_body(x_ref, o_ref, mval, midx):
    cb = pl.program_id(1)

    @pl.when(cb == 0)
    def _init():
        mval[...] = jnp.full((_R,), _NEG, jnp.float32)
        midx[...] = jnp.full((_R,), 0, jnp.int32)

    x = x_ref[...]
    ids = jax.lax.broadcasted_iota(jnp.int32, (_R, _C), 1) + cb * _C
    x = jnp.where(ids < _V, x, _NEG)
    bmax = jnp.max(x, axis=1)
    bidx = jnp.min(jnp.where(x == bmax[:, None], ids, _BIG), axis=1)
    better = bmax > mval[...]
    midx[...] = jnp.where(better, bidx, midx[...])
    mval[...] = jnp.where(better, bmax, mval[...])

    @pl.when(cb == _NC - 1)
    def _out():
        o_ref[0, 0, :] = midx[...]


def _rowwise_argmax(x):
    n = x.shape[0]
    nrb = n // _R
    out = pl.pallas_call(
        _argmax_body,
        grid=(nrb, _NC),
        in_specs=[pl.BlockSpec((_R, _C), lambda rb, cb: (rb, cb))],
        out_specs=pl.BlockSpec((1, 1, _R), lambda rb, cb: (rb, 0, 0)),
        out_shape=jax.ShapeDtypeStruct((nrb, 1, _R), jnp.int32),
        scratch_shapes=[
            pltpu.VMEM((_R,), jnp.float32),
            pltpu.VMEM((_R,), jnp.int32),
        ],
        compiler_params=pltpu.CompilerParams(
            dimension_semantics=("parallel", "arbitrary"),
        ),
    )(x)
    return out.reshape(n)


@jax.jit
def kernel(logits_ar, logits_parallel):
    b, ncm1, v = logits_parallel.shape
    token0 = _rowwise_argmax(logits_ar)
    rest = _rowwise_argmax(logits_parallel.reshape(b * ncm1, v))
    tokens = jnp.concatenate([token0[:, None], rest.reshape(b, ncm1)], axis=1)
    return tokens.astype(jnp.int32)
